# baseline (speedup 1.0000x reference)
import functools

import jax
import jax.numpy as jnp
from jax import lax
from jax.experimental import pallas as pl
from jax.experimental.pallas import tpu as pltpu
from jax.experimental.pallas import tpu_sc as plsc

VOCAB = 30522
HIDDEN = 768
MAX_POS = 512
BATCH = 1024
SEQ = 200
EPS = 1e-12

NC = 2
NS = 16
NW = NC * NS
TOK = BATCH * SEQ
TOK_W = TOK // NW
ROWS_W = TOK_W // SEQ
CHUNK = 40
PHASES = SEQ // CHUNK
LANES = 16
JV = HIDDEN // LANES


def _rsqrt16(x):
    i = lax.bitcast_convert_type(x, jnp.int32)
    i = jnp.int32(0x5F3759DF) - lax.shift_right_logical(i, 1)
    y = lax.bitcast_convert_type(i, jnp.float32)
    for _ in range(3):
        y = y * (1.5 - 0.5 * x * y * y)
    return y


def _sc_body(ids_hbm, word_hbm, pos_hbm, gamma_hbm, beta_hbm, out_hbm,
             idx_v, pos_v, g_v, b_v, rows_v, sem):
    wid = lax.axis_index("s") * NC + lax.axis_index("c")
    wbase = wid * TOK_W

    pltpu.sync_copy(ids_hbm.at[pl.ds(wbase, TOK_W)], idx_v)
    pltpu.sync_copy(gamma_hbm, g_v)
    pltpu.sync_copy(beta_hbm, b_v)

    def phase_body(p, _):
        pltpu.sync_copy(pos_hbm.at[pl.ds(p * CHUNK, CHUNK)], pos_v)

        def row_body(r, _):
            off = r * SEQ + p * CHUNK
            pltpu.async_copy(
                word_hbm.at[idx_v.at[pl.ds(off, CHUNK)]], rows_v, sem
            ).wait()

            def tok_body(t, _):
                acc = jnp.zeros((LANES,), jnp.float32)
                acc2 = jnp.zeros((LANES,), jnp.float32)
                for j in range(JV):
                    sl = pl.ds(j * LANES, LANES)
                    v = rows_v[t, sl] + pos_v[t, sl]
                    rows_v[t, sl] = v
                    acc = acc + v
                    acc2 = acc2 + v * v
                s = jnp.sum(acc)
                s2 = jnp.sum(acc2)
                mean = s * (1.0 / HIDDEN)
                var = jnp.maximum(s2 * (1.0 / HIDDEN) - mean * mean, 0.0)
                meanv = jnp.broadcast_to(mean, (LANES,))
                invv = _rsqrt16(jnp.broadcast_to(var + EPS, (LANES,)))
                for j in range(JV):
                    sl = pl.ds(j * LANES, LANES)
                    ag = invv * g_v[sl]
                    c = b_v[sl] - meanv * ag
                    rows_v[t, sl] = rows_v[t, sl] * ag + c
                return 0

            lax.fori_loop(0, CHUNK, tok_body, 0)
            pltpu.sync_copy(rows_v, out_hbm.at[pl.ds(wbase + off, CHUNK)])
            return 0

        lax.fori_loop(0, ROWS_W, row_body, 0)
        return 0

    lax.fori_loop(0, PHASES, phase_body, 0)


@jax.jit
def kernel(input_ids, word_emb, pos_emb, gamma, beta):
    ids_flat = input_ids.reshape(TOK).astype(jnp.int32)
    mesh = plsc.VectorSubcoreMesh(core_axis_name="c", subcore_axis_name="s")
    k = pl.kernel(
        _sc_body,
        out_type=jax.ShapeDtypeStruct((TOK, HIDDEN), jnp.float32),
        mesh=mesh,
        scratch_types=[
            pltpu.VMEM((TOK_W,), jnp.int32),
            pltpu.VMEM((CHUNK, HIDDEN), jnp.float32),
            pltpu.VMEM((HIDDEN,), jnp.float32),
            pltpu.VMEM((HIDDEN,), jnp.float32),
            pltpu.VMEM((CHUNK, HIDDEN), jnp.float32),
            pltpu.SemaphoreType.DMA,
        ],
        compiler_params=pltpu.CompilerParams(needs_layout_passes=False),
    )
    out = k(ids_flat, word_emb, pos_emb, gamma, beta)
    return out.reshape(BATCH, SEQ, HIDDEN)
